# baseline (speedup 1.0000x reference)
import functools
import math

import jax
import jax.numpy as jnp
import numpy as np
from jax import lax
from jax.experimental import pallas as pl
from jax.experimental.pallas import tpu as pltpu
from jax.experimental.pallas import tpu_sc as plsc

_VOCAB = 1000000
_EMB = 64
_BATCH = 1024
_SEQ = 200

_NC, _NS = 2, 16
_NW = _NC * _NS
_N = _BATCH * _SEQ
_K = 128
_ROWS_PER_W = _N // _NW
_CHUNKS = _ROWS_PER_W // _K
_LBLK = _BATCH // _K
_S = 10
_H = _S // 2
_GPC = _CHUNKS // _S


def _pe_table():
    position = np.arange(0, _SEQ, dtype=np.float64)[:, None]
    div_term = np.exp(
        np.arange(0, _EMB, 2, dtype=np.float64) * -(math.log(10000.0) / _EMB))
    pe = np.zeros((_SEQ, _EMB), dtype=np.float32)
    pe[:, 0::2] = np.sin(position * div_term).astype(np.float32)
    pe[:, 1::2] = np.cos(position * div_term).astype(np.float32)
    return pe


_PE = _pe_table()


def _sc_body(idx_hbm, table_hbm, out_hbm, idx_v, gbuf, *sems):
    gsem, osem = sems[:_S], sems[_S:]
    w = lax.axis_index("s") * _NC + lax.axis_index("c")
    pltpu.sync_copy(idx_hbm.at[w], idx_v)

    def gather(c, k):
        return pltpu.make_async_copy(
            table_hbm.at[idx_v.at[c]], gbuf.at[k], gsem[k])

    def wback(c, k):
        return pltpu.make_async_copy(gbuf.at[k], out_hbm.at[w, c], osem[k])

    for k in range(_H):
        gather(k, k).start()

    def turn(i, carry):
        for k in range(_S):
            t = i * _S + k
            kp = (k + _H) % _S
            cp = t + _H

            @pl.when(cp < _CHUNKS)
            def _():
                @pl.when(cp - _S >= 0)
                def _():
                    wback(cp - _S, kp).wait()
                gather(cp, kp).start()

            gather(t, k).wait()
            wback(t, k).start()
        return carry

    lax.fori_loop(0, _GPC, turn, 0)

    for c in range(_CHUNKS - _H, _CHUNKS):
        wback(c, c % _S).wait()


def _tc_body(x_ref, pe_ref, o_ref):
    x = x_ref[0]
    for eh in range(_EMB // 8):
        sl = x[:, :, 8 * eh:8 * eh + 8]
        t = jnp.transpose(sl, (0, 2, 1))
        pe8 = pe_ref[0, 0, 8 * eh:8 * eh + 8]
        o_ref[0, eh] = t + pe8.reshape(1, 8, 1)


def kernel(input, table):
    idx = input.T.reshape(_NW, _CHUNKS, _K)

    sc_gather = functools.partial(
        pl.kernel,
        out_type=jax.ShapeDtypeStruct((_NW, _CHUNKS, _K, _EMB), jnp.float32),
        mesh=plsc.VectorSubcoreMesh(
            core_axis_name="c", subcore_axis_name="s",
            num_cores=_NC, num_subcores=_NS),
        scratch_types=[
            pltpu.VMEM((_CHUNKS, _K), jnp.int32),
            pltpu.VMEM((_S, _K, _EMB), jnp.float32),
        ] + [pltpu.SemaphoreType.DMA] * (2 * _S),
        compiler_params=pltpu.CompilerParams(use_tc_tiling_on_sc=False),
    )(_sc_body)

    rows = sc_gather(idx, table)
    rows4 = rows.reshape(_SEQ, _LBLK, _K, _EMB)

    out5 = pl.pallas_call(
        _tc_body,
        grid=(_SEQ,),
        in_specs=[
            pl.BlockSpec((1, _LBLK, _K, _EMB), lambda l: (l, 0, 0, 0)),
            pl.BlockSpec((1, 1, _EMB), lambda l: (l, 0, 0)),
        ],
        out_specs=pl.BlockSpec((1, _EMB // 8, _LBLK, 8, _K),
                               lambda l: (l, 0, 0, 0, 0)),
        out_shape=jax.ShapeDtypeStruct((_SEQ, _EMB // 8, _LBLK, 8, _K),
                                       jnp.float32),
    )(rows4, _PE.reshape(_SEQ, 1, _EMB))

    return out5.transpose(0, 2, 4, 1, 3).reshape(_SEQ, _BATCH, _EMB)


# speedup vs baseline: 1.2132x; 1.2132x over previous
import functools
import math

import jax
import jax.numpy as jnp
import numpy as np
from jax import lax
from jax.experimental import pallas as pl
from jax.experimental.pallas import tpu as pltpu
from jax.experimental.pallas import tpu_sc as plsc

_VOCAB = 1000000
_EMB = 64
_BATCH = 1024
_SEQ = 200

_NC, _NS = 2, 16
_NW = _NC * _NS
_N = _BATCH * _SEQ
_K = 128
_ROWS_PER_W = _N // _NW
_CHUNKS = _ROWS_PER_W // _K
_LBLK = _BATCH // _K
_S = 10
_H = _S // 2
_GPC = _CHUNKS // _S


def _pe_table():
    position = np.arange(0, _SEQ, dtype=np.float64)[:, None]
    div_term = np.exp(
        np.arange(0, _EMB, 2, dtype=np.float64) * -(math.log(10000.0) / _EMB))
    pe = np.zeros((_SEQ, _EMB), dtype=np.float32)
    pe[:, 0::2] = np.sin(position * div_term).astype(np.float32)
    pe[:, 1::2] = np.cos(position * div_term).astype(np.float32)
    return pe


_PE = _pe_table()


def _sc_body(idx_hbm, table_hbm, out_hbm, idx_v, gbuf, *sems):
    gsem, osem = sems[:_S], sems[_S:]
    w = lax.axis_index("s") * _NC + lax.axis_index("c")
    pltpu.sync_copy(idx_hbm.at[w], idx_v)

    def gather(c, k):
        return pltpu.make_async_copy(
            table_hbm.at[idx_v.at[c]], gbuf.at[k], gsem[k])

    def wback(c, k):
        return pltpu.make_async_copy(gbuf.at[k], out_hbm.at[w, c], osem[k])

    for k in range(_H):
        gather(k, k).start()

    def turn(i, carry):
        for k in range(_S):
            t = i * _S + k
            kp = (k + _H) % _S
            cp = t + _H

            @pl.when(cp < _CHUNKS)
            def _():
                @pl.when(cp - _S >= 0)
                def _():
                    wback(cp - _S, kp).wait()
                gather(cp, kp).start()

            gather(t, k).wait()
            wback(t, k).start()
        return carry

    lax.fori_loop(0, _GPC, turn, 0)

    for c in range(_CHUNKS - _H, _CHUNKS):
        wback(c, c % _S).wait()


def _tc_body(x_ref, pe_ref, o_ref):
    o_ref[...] = x_ref[...] + pe_ref[...]


def kernel(input, table):
    idx = input.T.reshape(_NW, _CHUNKS, _K)

    sc_gather = functools.partial(
        pl.kernel,
        out_type=jax.ShapeDtypeStruct((_NW, _CHUNKS, _K, _EMB), jnp.float32),
        mesh=plsc.VectorSubcoreMesh(
            core_axis_name="c", subcore_axis_name="s",
            num_cores=_NC, num_subcores=_NS),
        scratch_types=[
            pltpu.VMEM((_CHUNKS, _K), jnp.int32),
            pltpu.VMEM((_S, _K, _EMB), jnp.float32),
        ] + [pltpu.SemaphoreType.DMA] * (2 * _S),
        compiler_params=pltpu.CompilerParams(use_tc_tiling_on_sc=False),
    )(_sc_body)

    rows = sc_gather(idx, table)
    rows3 = rows.reshape(_SEQ, _BATCH, _EMB)

    return pl.pallas_call(
        _tc_body,
        grid=(_SEQ,),
        in_specs=[
            pl.BlockSpec((1, _BATCH, _EMB), lambda l: (l, 0, 0)),
            pl.BlockSpec((1, 1, _EMB), lambda l: (l, 0, 0)),
        ],
        out_specs=pl.BlockSpec((1, _BATCH, _EMB), lambda l: (l, 0, 0)),
        out_shape=jax.ShapeDtypeStruct((_SEQ, _BATCH, _EMB), jnp.float32),
    )(rows3, _PE.reshape(_SEQ, 1, _EMB))


# speedup vs baseline: 1.4869x; 1.2256x over previous
import functools
import math

import jax
import jax.numpy as jnp
import numpy as np
from jax import lax
from jax.experimental import pallas as pl
from jax.experimental.pallas import tpu as pltpu
from jax.experimental.pallas import tpu_sc as plsc

_VOCAB = 1000000
_EMB = 64
_BATCH = 1024
_SEQ = 200

_NC, _NS = 2, 16
_NW = _NC * _NS
_N = _BATCH * _SEQ
_K = 128
_ROWS_PER_W = _N // _NW
_CHUNKS = _ROWS_PER_W // _K
_S = 10
_H = _S // 2
_GPC = _CHUNKS // _S
_V = 16


def _pe_table():
    position = np.arange(0, _SEQ, dtype=np.float64)[:, None]
    div_term = np.exp(
        np.arange(0, _EMB, 2, dtype=np.float64) * -(math.log(10000.0) / _EMB))
    pe = np.zeros((_SEQ, _EMB), dtype=np.float32)
    pe[:, 0::2] = np.sin(position * div_term).astype(np.float32)
    pe[:, 1::2] = np.cos(position * div_term).astype(np.float32)
    return pe


_PE_CHUNK = _pe_table()[
    (np.arange(_NW * _CHUNKS) // (_BATCH // _K)).reshape(_NW, _CHUNKS)]


def _sc_body(idx_hbm, table_hbm, pe_hbm, out_hbm, idx_v, pe_v, gbuf, *sems):
    gsem, osem = sems[:_S], sems[_S:]
    w = lax.axis_index("s") * _NC + lax.axis_index("c")
    pltpu.sync_copy(idx_hbm.at[w], idx_v)
    pltpu.sync_copy(pe_hbm.at[w], pe_v)

    def gather(c, k):
        return pltpu.make_async_copy(
            table_hbm.at[idx_v.at[c]], gbuf.at[k], gsem[k])

    def wback(c, k):
        return pltpu.make_async_copy(gbuf.at[k], out_hbm.at[w, c], osem[k])

    def pe_add(t, k):
        pev = [pe_v[t, _V * j:_V * (j + 1)] for j in range(_EMB // _V)]

        def row(r, carry):
            for j in range(_EMB // _V):
                sl = slice(_V * j, _V * (j + 1))
                gbuf[k, r, sl] = gbuf[k, r, sl] + pev[j]
            return carry

        lax.fori_loop(0, _K, row, 0)

    for k in range(_H):
        gather(k, k).start()

    def turn(i, carry):
        for k in range(_S):
            t = i * _S + k
            kp = (k + _H) % _S
            cp = t + _H

            @pl.when(cp < _CHUNKS)
            def _():
                @pl.when(cp - _S >= 0)
                def _():
                    wback(cp - _S, kp).wait()
                gather(cp, kp).start()

            gather(t, k).wait()
            pe_add(t, k)
            wback(t, k).start()
        return carry

    lax.fori_loop(0, _GPC, turn, 0)

    for c in range(_CHUNKS - _H, _CHUNKS):
        wback(c, c % _S).wait()


def kernel(input, table):
    idx = input.T.reshape(_NW, _CHUNKS, _K)

    sc_gather = functools.partial(
        pl.kernel,
        out_type=jax.ShapeDtypeStruct((_NW, _CHUNKS, _K, _EMB), jnp.float32),
        mesh=plsc.VectorSubcoreMesh(
            core_axis_name="c", subcore_axis_name="s",
            num_cores=_NC, num_subcores=_NS),
        scratch_types=[
            pltpu.VMEM((_CHUNKS, _K), jnp.int32),
            pltpu.VMEM((_CHUNKS, _EMB), jnp.float32),
            pltpu.VMEM((_S, _K, _EMB), jnp.float32),
        ] + [pltpu.SemaphoreType.DMA] * (2 * _S),
        compiler_params=pltpu.CompilerParams(use_tc_tiling_on_sc=False),
    )(_sc_body)

    rows = sc_gather(idx, table, jnp.asarray(_PE_CHUNK))
    return rows.reshape(_SEQ, _BATCH, _EMB)


# speedup vs baseline: 1.4885x; 1.0011x over previous
import functools
import math

import jax
import jax.numpy as jnp
import numpy as np
from jax import lax
from jax.experimental import pallas as pl
from jax.experimental.pallas import tpu as pltpu
from jax.experimental.pallas import tpu_sc as plsc

_VOCAB = 1000000
_EMB = 64
_BATCH = 1024
_SEQ = 200

_NC, _NS = 2, 16
_NW = _NC * _NS
_N = _BATCH * _SEQ
_K = 128
_ROWS_PER_W = _N // _NW
_CHUNKS = _ROWS_PER_W // _K
_TOTC = _NW * _CHUNKS
_S = 10
_H = _S // 2
_GPC = _CHUNKS // _S
_V = 16


def _pe_table():
    position = np.arange(0, _SEQ, dtype=np.float64)[:, None]
    div_term = np.exp(
        np.arange(0, _EMB, 2, dtype=np.float64) * -(math.log(10000.0) / _EMB))
    pe = np.zeros((_SEQ, _EMB), dtype=np.float32)
    pe[:, 0::2] = np.sin(position * div_term).astype(np.float32)
    pe[:, 1::2] = np.cos(position * div_term).astype(np.float32)
    return pe


_PE_CHUNK = _pe_table()[
    (np.arange(_TOTC) // (_BATCH // _K)).reshape(_NW, _CHUNKS)]


def _sc_body(idx_hbm, table_hbm, pe_hbm, out_hbm, idx_v, pe_v, gbuf, *sems):
    gsem, osem = sems[:_S], sems[_S:]
    w = lax.axis_index("s") * _NC + lax.axis_index("c")
    pltpu.sync_copy(idx_hbm.at[w], idx_v)
    pltpu.sync_copy(pe_hbm.at[w], pe_v)

    def gather(c, k):
        return pltpu.make_async_copy(
            table_hbm.at[idx_v.at[c]], gbuf.at[k], gsem[k])

    def wback(c, k):
        return pltpu.make_async_copy(gbuf.at[k], out_hbm.at[w, c], osem[k])

    def pe_add(t, k):
        pev = [pe_v[t, _V * j:_V * (j + 1)] for j in range(_EMB // _V)]

        def row(r, carry):
            for j in range(_EMB // _V):
                sl = slice(_V * j, _V * (j + 1))
                gbuf[k, r, sl] = gbuf[k, r, sl] + pev[j]
            return carry

        lax.fori_loop(0, _K, row, 0)

    for k in range(_H):
        gather(k, k).start()

    def turn(i, carry):
        for k in range(_S):
            t = i * _S + k
            kp = (k + _H) % _S
            cp = t + _H

            @pl.when(cp < _CHUNKS)
            def _():
                @pl.when(cp - _S >= 0)
                def _():
                    wback(cp - _S, kp).wait()
                gather(cp, kp).start()

            gather(t, k).wait()
            pe_add(t, k)
            wback(t, k).start()
        return carry

    lax.fori_loop(0, _GPC, turn, 0)

    for c in range(_CHUNKS - _H, _CHUNKS):
        wback(c, c % _S).wait()


def kernel(input, table):
    idx = input.T.reshape(_NW, _CHUNKS, _K)

    sc_gather = functools.partial(
        pl.kernel,
        out_type=jax.ShapeDtypeStruct((_NW, _CHUNKS, _K, _EMB), jnp.float32),
        mesh=plsc.VectorSubcoreMesh(
            core_axis_name="c", subcore_axis_name="s",
            num_cores=_NC, num_subcores=_NS),
        scratch_types=[
            pltpu.VMEM((_CHUNKS, _K), jnp.int32),
            pltpu.VMEM((_CHUNKS, _EMB), jnp.float32),
            pltpu.VMEM((_S, _K, _EMB), jnp.float32),
        ] + [pltpu.SemaphoreType.DMA] * (2 * _S),
        compiler_params=pltpu.CompilerParams(use_tc_tiling_on_sc=False),
    )(_sc_body)

    rows = sc_gather(idx, table, jnp.asarray(_PE_CHUNK))
    return rows.reshape(_SEQ, _BATCH, _EMB)
